# baseline (speedup 1.0000x reference)
import functools

import jax
import jax.numpy as jnp
from jax import lax
from jax.experimental import pallas as pl
from jax.experimental.pallas import tpu as pltpu
from jax.experimental.pallas import tpu_sc as plsc

BATCH = 16384
DEPTH = 1000
NUM_CORES = 2
NUM_SUBCORES = 16
NUM_WORKERS = NUM_CORES * NUM_SUBCORES
PER_W = BATCH // NUM_WORKERS
R = 16
CHUNKS = PER_W // R
LANES = 16
ROW_OFFS = tuple(range(0, DEPTH - LANES, LANES)) + (DEPTH - LANES,)


def _one_hot_body(idx_hbm, out_hbm, idx_v, buf0, buf1, sem0, sem1):
    wid = lax.axis_index("s") * NUM_CORES + lax.axis_index("c")
    base = wid * PER_W

    pltpu.sync_copy(idx_hbm.at[pl.ds(base * 1, PER_W)], idx_v)

    zeros16 = jnp.zeros((LANES,), jnp.float32)

    def _zero(r, _):
        for off in ROW_OFFS:
            buf0[r, pl.ds(off, LANES)] = zeros16
            buf1[r, pl.ds(off, LANES)] = zeros16
        return _

    lax.fori_loop(0, R, _zero, None)

    bufs = (buf0, buf1)
    sems = (sem0, sem1)
    ones16 = jnp.full((LANES,), 1.0, jnp.float32)
    iota16 = lax.iota(jnp.int32, LANES)
    groups = R // LANES

    def scatter_chunk(buf, c, vals):
        for g in range(groups):
            idx16 = idx_v[pl.ds(c * R + g * LANES, LANES)]
            rows16 = g * LANES + iota16
            plsc.store_scatter(buf, [rows16, idx16], vals)

    handles = [None, None]
    for c in range(CHUNKS):
        b = c % 2
        if handles[b] is not None:
            handles[b].wait()
            scatter_chunk(bufs[b], c - 2, zeros16)
        scatter_chunk(bufs[b], c, ones16)
        handles[b] = pltpu.async_copy(
            bufs[b],
            out_hbm.at[pl.ds(base + c * R, R)],
            sems[b],
        )
    handles[0].wait()
    handles[1].wait()


@functools.partial(jax.jit, static_argnames=())
def _one_hot_sc(idx):
    mesh = plsc.VectorSubcoreMesh(core_axis_name="c", subcore_axis_name="s")
    k = functools.partial(
        pl.kernel,
        mesh=mesh,
        out_type=jax.ShapeDtypeStruct((BATCH, DEPTH), jnp.float32),
        scratch_types=[
            pltpu.VMEM((PER_W,), jnp.int32),
            pltpu.VMEM((R, DEPTH), jnp.float32),
            pltpu.VMEM((R, DEPTH), jnp.float32),
            pltpu.SemaphoreType.DMA,
            pltpu.SemaphoreType.DMA,
        ],
        compiler_params=pltpu.CompilerParams(
            needs_layout_passes=False,
            use_tc_tiling_on_sc=True,
        ),
    )(_one_hot_body)
    return k(idx)


def kernel(X_in, ones):
    del ones
    return _one_hot_sc(X_in.astype(jnp.int32))


# speedup vs baseline: 2.2153x; 2.2153x over previous
import functools

import jax
import jax.numpy as jnp
from jax import lax
from jax.experimental import pallas as pl
from jax.experimental.pallas import tpu as pltpu
from jax.experimental.pallas import tpu_sc as plsc

BATCH = 16384
DEPTH = 1000
NUM_CORES = 2
NUM_SUBCORES = 16
NUM_WORKERS = NUM_CORES * NUM_SUBCORES
PER_W = BATCH // NUM_WORKERS
CW = 128
CHUNKS = PER_W // CW
LANES = 16
GROUPS = CW // LANES


def _one_hot_body(idx_hbm, out_hbm, idx_v, buf, sem):
    wid = lax.axis_index("s") * NUM_CORES + lax.axis_index("c")
    base = wid * PER_W

    pltpu.sync_copy(idx_hbm.at[pl.ds(base * 1, PER_W)], idx_v)

    zeros16 = jnp.zeros((LANES,), jnp.float32)

    def _zero(r, _):
        for u in range(GROUPS):
            buf[r, pl.ds(u * LANES, LANES)] = zeros16
        return _

    lax.fori_loop(0, DEPTH, _zero, None)

    ones16 = jnp.full((LANES,), 1.0, jnp.float32)
    iota16 = lax.iota(jnp.int32, LANES)

    def scatter_stripe(c, vals):
        for g in range(GROUPS):
            idx16 = idx_v[pl.ds(c * CW + g * LANES, LANES)]
            cols16 = g * LANES + iota16
            plsc.store_scatter(buf, [idx16, cols16], vals)

    for c in range(CHUNKS):
        scatter_stripe(c, ones16)
        pltpu.async_copy(
            buf,
            out_hbm.at[:, pl.ds(base + c * CW, CW)],
            sem,
        ).wait()
        if c + 1 < CHUNKS:
            scatter_stripe(c, zeros16)


@functools.partial(jax.jit, static_argnames=())
def _one_hot_sc(idx):
    mesh = plsc.VectorSubcoreMesh(core_axis_name="c", subcore_axis_name="s")
    k = functools.partial(
        pl.kernel,
        mesh=mesh,
        out_type=jax.ShapeDtypeStruct((DEPTH, BATCH), jnp.float32),
        scratch_types=[
            pltpu.VMEM((PER_W,), jnp.int32),
            pltpu.VMEM((DEPTH, CW), jnp.float32),
            pltpu.SemaphoreType.DMA,
        ],
        compiler_params=pltpu.CompilerParams(
            needs_layout_passes=False,
            use_tc_tiling_on_sc=True,
        ),
    )(_one_hot_body)
    return k(idx).T


def kernel(X_in, ones):
    del ones
    return _one_hot_sc(X_in.astype(jnp.int32))


# speedup vs baseline: 2.2301x; 1.0067x over previous
import functools

import jax
import jax.numpy as jnp
from jax import lax
from jax.experimental import pallas as pl
from jax.experimental.pallas import tpu as pltpu
from jax.experimental.pallas import tpu_sc as plsc

BATCH = 16384
DEPTH = 1000
NUM_CORES = 2
NUM_SUBCORES = 16
NUM_WORKERS = NUM_CORES * NUM_SUBCORES
PER_W = BATCH // NUM_WORKERS
CW = 128
CHUNKS = PER_W // CW
LANES = 16
GROUPS = CW // LANES


def _one_hot_body(idx_hbm, out_hbm, idx_v, buf, sem, isem):
    wid = lax.axis_index("s") * NUM_CORES + lax.axis_index("c")
    base = wid * PER_W

    idx_cp = pltpu.async_copy(
        idx_hbm.at[pl.ds(base * 1, PER_W)], idx_v, isem
    )

    zeros16 = jnp.zeros((LANES,), jnp.float32)

    def _zero(r, _):
        for u in range(GROUPS):
            buf[r, pl.ds(u * LANES, LANES)] = zeros16
        return _

    lax.fori_loop(0, DEPTH, _zero, None)
    idx_cp.wait()

    ones16 = jnp.full((LANES,), 1.0, jnp.float32)
    iota16 = lax.iota(jnp.int32, LANES)

    def scatter_stripe(c, vals):
        for g in range(GROUPS):
            idx16 = idx_v[pl.ds(c * CW + g * LANES, LANES)]
            cols16 = g * LANES + iota16
            plsc.store_scatter(buf, [idx16, cols16], vals)

    for c in range(CHUNKS):
        scatter_stripe(c, ones16)
        pltpu.async_copy(
            buf,
            out_hbm.at[:, pl.ds(base + c * CW, CW)],
            sem,
        ).wait()
        if c + 1 < CHUNKS:
            scatter_stripe(c, zeros16)


@functools.partial(jax.jit, static_argnames=())
def _one_hot_sc(idx):
    mesh = plsc.VectorSubcoreMesh(core_axis_name="c", subcore_axis_name="s")
    k = functools.partial(
        pl.kernel,
        mesh=mesh,
        out_type=jax.ShapeDtypeStruct((DEPTH, BATCH), jnp.float32),
        scratch_types=[
            pltpu.VMEM((PER_W,), jnp.int32),
            pltpu.VMEM((DEPTH, CW), jnp.float32),
            pltpu.SemaphoreType.DMA,
            pltpu.SemaphoreType.DMA,
        ],
        compiler_params=pltpu.CompilerParams(
            needs_layout_passes=False,
            use_tc_tiling_on_sc=True,
        ),
    )(_one_hot_body)
    return k(idx).T


def kernel(X_in, ones):
    del ones
    return _one_hot_sc(X_in.astype(jnp.int32))


# speedup vs baseline: 2.3442x; 1.0511x over previous
import functools

import jax
import jax.numpy as jnp
from jax import lax
from jax.experimental import pallas as pl
from jax.experimental.pallas import tpu as pltpu
from jax.experimental.pallas import tpu_sc as plsc

BATCH = 16384
DEPTH = 1000
NUM_CORES = 2
NUM_SUBCORES = 16
NUM_WORKERS = NUM_CORES * NUM_SUBCORES
PER_W = BATCH // NUM_WORKERS
CW = 128
CHUNKS = PER_W // CW
LANES = 16
GROUPS = CW // LANES
HSPLIT = 504


def _one_hot_body(idx_hbm, out_hbm, idx_v, buf, sem, isem):
    wid = lax.axis_index("s") * NUM_CORES + lax.axis_index("c")
    base = wid * PER_W

    idx_cp = pltpu.async_copy(
        idx_hbm.at[pl.ds(base * 1, PER_W)], idx_v, isem
    )

    zeros16 = jnp.zeros((LANES,), jnp.float32)
    ones16 = jnp.full((LANES,), 1.0, jnp.float32)
    iota16 = lax.iota(jnp.int32, LANES)

    def _zero(r, _):
        for u in range(GROUPS):
            buf[r, pl.ds(u * LANES, LANES)] = zeros16
        return _

    def scatter_stripe(c, vals):
        for g in range(GROUPS):
            idx16 = idx_v[pl.ds(c * CW + g * LANES, LANES)]
            cols16 = g * LANES + iota16
            plsc.store_scatter(buf, [idx16, cols16], vals)

    def scatter_half(c, top):
        for g in range(GROUPS):
            idx16 = idx_v[pl.ds(c * CW + g * LANES, LANES)]
            cols16 = g * LANES + iota16
            in_half = (idx16 < HSPLIT) if top else (idx16 >= HSPLIT)
            vals = jnp.where(in_half, ones16, zeros16)
            plsc.store_scatter(buf, [idx16, cols16], vals)

    def stripe_dma(c, lo, nrows):
        return pltpu.async_copy(
            buf.at[pl.ds(lo, nrows)],
            out_hbm.at[pl.ds(lo, nrows), pl.ds(base + c * CW, CW)],
            sem,
        )

    lax.fori_loop(0, HSPLIT, _zero, None)
    idx_cp.wait()
    scatter_half(0, top=True)
    top = stripe_dma(0, 0, HSPLIT)
    lax.fori_loop(HSPLIT, DEPTH, _zero, None)
    top.wait()
    scatter_half(0, top=False)
    stripe_dma(0, HSPLIT, DEPTH - HSPLIT).wait()
    scatter_stripe(0, zeros16)

    for c in range(1, CHUNKS):
        scatter_stripe(c, ones16)
        stripe_dma(c, 0, DEPTH).wait()
        if c + 1 < CHUNKS:
            scatter_stripe(c, zeros16)


@functools.partial(jax.jit, static_argnames=())
def _one_hot_sc(idx):
    mesh = plsc.VectorSubcoreMesh(core_axis_name="c", subcore_axis_name="s")
    k = functools.partial(
        pl.kernel,
        mesh=mesh,
        out_type=jax.ShapeDtypeStruct((DEPTH, BATCH), jnp.float32),
        scratch_types=[
            pltpu.VMEM((PER_W,), jnp.int32),
            pltpu.VMEM((DEPTH, CW), jnp.float32),
            pltpu.SemaphoreType.DMA,
            pltpu.SemaphoreType.DMA,
        ],
        compiler_params=pltpu.CompilerParams(
            needs_layout_passes=False,
            use_tc_tiling_on_sc=True,
        ),
    )(_one_hot_body)
    return k(idx).T


def kernel(X_in, ones):
    del ones
    return _one_hot_sc(X_in.astype(jnp.int32))


# speedup vs baseline: 2.4006x; 1.0241x over previous
import functools

import jax
import jax.numpy as jnp
from jax import lax
from jax.experimental import pallas as pl
from jax.experimental.pallas import tpu as pltpu
from jax.experimental.pallas import tpu_sc as plsc

BATCH = 16384
DEPTH = 1000
NUM_CORES = 2
NUM_SUBCORES = 16
NUM_WORKERS = NUM_CORES * NUM_SUBCORES
PER_W = BATCH // NUM_WORKERS
CW = 128
CHUNKS = PER_W // CW
LANES = 16
GROUPS = CW // LANES
QBOUNDS = (0, 248, 504, 752, 1000)


def _one_hot_body(idx_hbm, out_hbm, idx_v, buf, sem, isem):
    wid = lax.axis_index("s") * NUM_CORES + lax.axis_index("c")
    base = wid * PER_W

    idx_cp = pltpu.async_copy(
        idx_hbm.at[pl.ds(base * 1, PER_W)], idx_v, isem
    )

    zeros16 = jnp.zeros((LANES,), jnp.float32)
    ones16 = jnp.full((LANES,), 1.0, jnp.float32)
    iota16 = lax.iota(jnp.int32, LANES)

    def _zero(r, _):
        for u in range(GROUPS):
            buf[r, pl.ds(u * LANES, LANES)] = zeros16
        return _

    def scatter_stripe(c, vals):
        for g in range(GROUPS):
            idx16 = idx_v[pl.ds(c * CW + g * LANES, LANES)]
            cols16 = g * LANES + iota16
            plsc.store_scatter(buf, [idx16, cols16], vals)

    def scatter_range(c, lo, hi):
        for g in range(GROUPS):
            idx16 = idx_v[pl.ds(c * CW + g * LANES, LANES)]
            cols16 = g * LANES + iota16
            in_range = (idx16 >= lo) & (idx16 < hi)
            vals = jnp.where(in_range, ones16, zeros16)
            plsc.store_scatter(buf, [idx16, cols16], vals)

    def stripe_dma(c, lo, nrows):
        return pltpu.async_copy(
            buf.at[pl.ds(lo, nrows)],
            out_hbm.at[pl.ds(lo, nrows), pl.ds(base + c * CW, CW)],
            sem,
        )

    lax.fori_loop(0, QBOUNDS[1], _zero, None)
    idx_cp.wait()
    prev = None
    for q in range(len(QBOUNDS) - 1):
        lo, hi = QBOUNDS[q], QBOUNDS[q + 1]
        if prev is not None:
            prev.wait()
        scatter_range(0, lo, hi)
        prev = stripe_dma(0, lo, hi - lo)
        if q + 2 < len(QBOUNDS):
            lax.fori_loop(QBOUNDS[q + 1], QBOUNDS[q + 2], _zero, None)
    prev.wait()
    scatter_stripe(0, zeros16)

    for c in range(1, CHUNKS):
        scatter_stripe(c, ones16)
        stripe_dma(c, 0, DEPTH).wait()
        if c + 1 < CHUNKS:
            scatter_stripe(c, zeros16)


@functools.partial(jax.jit, static_argnames=())
def _one_hot_sc(idx):
    mesh = plsc.VectorSubcoreMesh(core_axis_name="c", subcore_axis_name="s")
    k = functools.partial(
        pl.kernel,
        mesh=mesh,
        out_type=jax.ShapeDtypeStruct((DEPTH, BATCH), jnp.float32),
        scratch_types=[
            pltpu.VMEM((PER_W,), jnp.int32),
            pltpu.VMEM((DEPTH, CW), jnp.float32),
            pltpu.SemaphoreType.DMA,
            pltpu.SemaphoreType.DMA,
        ],
        compiler_params=pltpu.CompilerParams(
            needs_layout_passes=False,
            use_tc_tiling_on_sc=True,
        ),
    )(_one_hot_body)
    return k(idx).T


def kernel(X_in, ones):
    del ones
    return _one_hot_sc(X_in.astype(jnp.int32))
